# baseline (speedup 1.0000x reference)
import functools

import jax
import jax.numpy as jnp
from jax import lax
from jax.experimental import pallas as pl
from jax.experimental.pallas import tpu as pltpu
from jax.experimental.pallas import tpu_sc as plsc

B = 16384
K = 8192
D = 32
BETA = 0.25

BN = 256
BKC = 128
GRID = B // BN

NW = 32
CHUNK = B // NW
GSUB = 128
NG = CHUNK // GSUB


def _tree_min(parts):
    while len(parts) > 1:
        parts = [jnp.minimum(parts[i], parts[i + 1])
                 for i in range(0, len(parts), 2)]
    return parts[0]


def _argmin_body(lt_ref, w_ref, idx_ref, md_ref, w2_ref):
    @pl.when(pl.program_id(0) == 0)
    def _():
        wf = w_ref[...]
        w2_ref[...] = jnp.sum(wf * wf, axis=1, keepdims=True)
    lt = lt_ref[...]
    l2 = jnp.sum(lt * lt, axis=0, keepdims=True)
    lt2 = lt + lt
    s_iota = lax.broadcasted_iota(jnp.int32, (8, BN), 0).astype(jnp.float32)
    NGRP = BKC // 8

    def step(c, carry):
        bestv8, besti8 = carry
        wc = w_ref[pl.ds(c * BKC, BKC), :]
        w2 = w2_ref[pl.ds(c * BKC, BKC), :]
        mm2 = lax.dot_general(wc, lt2, (((1,), (0,)), ((), ())),
                              preferred_element_type=jnp.float32)
        dist = (l2 + w2) - mm2
        parts = [lax.slice_in_dim(dist, g * 8, (g + 1) * 8, axis=0)
                 for g in range(NGRP)]
        vals = parts
        idxs = [None] * NGRP
        first = True
        while len(vals) > 1:
            nv, ni = [], []
            for i in range(0, len(vals), 2):
                a, b = vals[i], vals[i + 1]
                le = a <= b
                nv.append(jnp.minimum(a, b))
                if first:
                    ni.append(jnp.where(le, float(i), float(i + 1)))
                else:
                    ni.append(jnp.where(le, idxs[i], idxs[i + 1]))
            vals, idxs, first = nv, ni, False
        r8, gm = vals[0], idxs[0]
        k8 = gm * 8.0 + (s_iota + c * float(BKC))
        upd = r8 < bestv8
        return (jnp.where(upd, r8, bestv8), jnp.where(upd, k8, besti8))

    carry = (jnp.full((8, BN), jnp.inf, jnp.float32),
             jnp.zeros((8, BN), jnp.float32))
    for c in range(K // BKC):
        carry = step(c, carry)
    bestv8, besti8 = carry
    bv = jnp.min(bestv8, axis=0, keepdims=True)
    cand = jnp.where(bestv8 == bv, besti8, float(2 * K))
    bi = jnp.min(cand, axis=0, keepdims=True)
    idx_ref[...] = bi.astype(jnp.int32).reshape(1, 1, BN)
    md_ref[...] = bv.reshape(1, 1, BN)


_argmin_call = pl.pallas_call(
    _argmin_body,
    grid=(GRID,),
    in_specs=[
        pl.BlockSpec((D, BN), lambda i: (0, i)),
        pl.BlockSpec((K, D), lambda i: (0, 0)),
    ],
    out_specs=[
        pl.BlockSpec((1, 1, BN), lambda i: (i, 0, 0)),
        pl.BlockSpec((1, 1, BN), lambda i: (i, 0, 0)),
    ],
    out_shape=[
        jax.ShapeDtypeStruct((GRID, 1, BN), jnp.int32),
        jax.ShapeDtypeStruct((GRID, 1, BN), jnp.float32),
    ],
    scratch_shapes=[pltpu.VMEM((K, 1), jnp.float32)],
    compiler_params=pltpu.CompilerParams(
        dimension_semantics=("arbitrary",)),
)


def _sc_body(inds_hbm, w_hbm, ql_hbm, cnt_hbm,
             idx_v, rows_v, cnt_v, sem):
    wid = lax.axis_index("s") * 2 + lax.axis_index("c")

    pltpu.sync_copy(inds_hbm.at[wid], idx_v)
    cps = [pltpu.async_copy(w_hbm.at[idx_v.at[j]], rows_v.at[j], sem)
           for j in range(NG)]

    def zero_step(z, _):
        cnt_v[pl.ds(z * 16, 16)] = jnp.zeros((16,), jnp.float32)
        return 0

    lax.fori_loop(0, K // 16, zero_step, 0, unroll=8)
    ones = jnp.ones((16,), jnp.float32)
    for j in range(NG):
        for c in range(GSUB // 16):
            iv = idx_v[j, pl.ds(c * 16, 16)]
            plsc.addupdate_scatter(cnt_v, [iv], ones)
    pltpu.sync_copy(cnt_v, cnt_hbm.at[wid])

    for cp in cps:
        cp.wait()
    pltpu.sync_copy(rows_v, ql_hbm.at[wid])


@functools.cache
def _sc_call():
    return pl.kernel(
        _sc_body,
        out_type=[
            jax.ShapeDtypeStruct((NW, NG, GSUB, D), jnp.float32),
            jax.ShapeDtypeStruct((NW, K), jnp.float32),
        ],
        mesh=plsc.VectorSubcoreMesh(core_axis_name="c",
                                    subcore_axis_name="s"),
        scratch_types=[
            pltpu.VMEM((NG, GSUB), jnp.int32),
            pltpu.VMEM((NG, GSUB, D), jnp.float32),
            pltpu.VMEM((K,), jnp.float32),
            pltpu.SemaphoreType.DMA,
        ],
        compiler_params=pltpu.CompilerParams(needs_layout_passes=False,
                                             use_tc_tiling_on_sc=False),
    )


def _final_body(cnt_ref, q_ref, lat_ref, md_ref, vq_ref, ent_ref, cm_ref):
    def ss_step(j, acc):
        q = q_ref[pl.ds(j * 512, 512), :]
        lv = lat_ref[pl.ds(j * 512, 512), :]
        diff = q - lv
        return acc + jnp.sum(diff * diff)

    ssum = lax.fori_loop(0, B // 512, ss_step, jnp.float32(0.0))
    m = ssum * (1.0 / (B * D))
    vq_ref[0, 0] = m * BETA + m

    def ent_step(j, acc):
        c = cnt_ref[:, pl.ds(j * 512, 512)]
        p = jnp.sum(c, axis=0, keepdims=True) * (1.0 / B)
        return acc + jnp.sum(p * jnp.log(p + 1e-10))

    ent = lax.fori_loop(0, K // 512, ent_step, jnp.float32(0.0))
    ent_ref[0, 0] = -ent
    cm_ref[0, 0] = jnp.sum(md_ref[...]) * (1.0 / B)


_final_call = pl.pallas_call(
    _final_body,
    in_specs=[
        pl.BlockSpec(memory_space=pltpu.VMEM),
        pl.BlockSpec(memory_space=pltpu.VMEM),
        pl.BlockSpec(memory_space=pltpu.VMEM),
        pl.BlockSpec(memory_space=pltpu.VMEM),
    ],
    out_specs=[
        pl.BlockSpec(memory_space=pltpu.SMEM),
        pl.BlockSpec(memory_space=pltpu.SMEM),
        pl.BlockSpec(memory_space=pltpu.SMEM),
    ],
    out_shape=[jax.ShapeDtypeStruct((1, 1), jnp.float32)] * 3,
)


def kernel(latents, W):
    lt = latents.T
    idx3, md3 = _argmin_call(lt, W)
    inds = idx3.reshape(NW, NG, GSUB)
    q4, counts = _sc_call()(inds, W)
    quantized = q4.reshape(B, D)
    vq, ent, cm = _final_call(counts, quantized, latents, md3)
    encoding_inds = idx3.reshape(B, 1)
    return (quantized, vq[0, 0], ent[0, 0], encoding_inds, cm[0, 0])


# speedup vs baseline: 1.0971x; 1.0971x over previous
import functools

import jax
import jax.numpy as jnp
from jax import lax
from jax.experimental import pallas as pl
from jax.experimental.pallas import tpu as pltpu
from jax.experimental.pallas import tpu_sc as plsc

B = 16384
K = 8192
D = 32
BETA = 0.25

BN = 256
BKC = 128
GRID = B // BN

NW = 32
CHUNK = B // NW
GSUB = 128
NG = CHUNK // GSUB


def _tree_min(parts):
    while len(parts) > 1:
        parts = [jnp.minimum(parts[i], parts[i + 1])
                 for i in range(0, len(parts), 2)]
    return parts[0]


def _argmin_body(lt_ref, w_ref, idx_ref, md_ref, w2_ref):
    @pl.when(pl.program_id(0) == 0)
    def _():
        wf = w_ref[...]
        w2_ref[...] = jnp.sum(wf * wf, axis=1, keepdims=True)
    lt = lt_ref[...]
    l2 = jnp.sum(lt * lt, axis=0, keepdims=True)
    lt2 = lt + lt
    s_iota = lax.broadcasted_iota(jnp.int32, (8, BN), 0).astype(jnp.float32)
    NGRP = BKC // 8

    def step(c, carry):
        bestv8, besti8 = carry
        wc = w_ref[pl.ds(c * BKC, BKC), :]
        w2 = w2_ref[pl.ds(c * BKC, BKC), :]
        mm2 = lax.dot_general(wc, lt2, (((1,), (0,)), ((), ())),
                              preferred_element_type=jnp.float32)
        dist = (l2 + w2) - mm2
        parts = [lax.slice_in_dim(dist, g * 8, (g + 1) * 8, axis=0)
                 for g in range(NGRP)]
        vals = parts
        idxs = [None] * NGRP
        first = True
        while len(vals) > 1:
            nv, ni = [], []
            for i in range(0, len(vals), 2):
                a, b = vals[i], vals[i + 1]
                le = a <= b
                nv.append(jnp.minimum(a, b))
                if first:
                    ni.append(jnp.where(le, float(i), float(i + 1)))
                else:
                    ni.append(jnp.where(le, idxs[i], idxs[i + 1]))
            vals, idxs, first = nv, ni, False
        r8, gm = vals[0], idxs[0]
        k8 = gm * 8.0 + (s_iota + c * float(BKC))
        upd = r8 < bestv8
        return (jnp.where(upd, r8, bestv8), jnp.where(upd, k8, besti8))

    carry = (jnp.full((8, BN), jnp.inf, jnp.float32),
             jnp.zeros((8, BN), jnp.float32))
    for c in range(K // BKC):
        carry = step(c, carry)
    bestv8, besti8 = carry
    bv = jnp.min(bestv8, axis=0, keepdims=True)
    cand = jnp.where(bestv8 == bv, besti8, float(2 * K))
    bi = jnp.min(cand, axis=0, keepdims=True)
    idx_ref[...] = bi.astype(jnp.int32).reshape(1, 1, BN)
    md_ref[...] = bv.reshape(1, 1, BN)


_argmin_call = pl.pallas_call(
    _argmin_body,
    grid=(GRID,),
    in_specs=[
        pl.BlockSpec((D, BN), lambda i: (0, i)),
        pl.BlockSpec((K, D), lambda i: (0, 0)),
    ],
    out_specs=[
        pl.BlockSpec((1, 1, BN), lambda i: (i, 0, 0)),
        pl.BlockSpec((1, 1, BN), lambda i: (i, 0, 0)),
    ],
    out_shape=[
        jax.ShapeDtypeStruct((GRID, 1, BN), jnp.int32),
        jax.ShapeDtypeStruct((GRID, 1, BN), jnp.float32),
    ],
    scratch_shapes=[pltpu.VMEM((K, 1), jnp.float32)],
    compiler_params=pltpu.CompilerParams(
        dimension_semantics=("arbitrary",)),
)


def _sc_body(inds_hbm, w_hbm, ql_hbm, cnt_hbm,
             idx_v, rows_v, cnt_v, sem):
    wid = lax.axis_index("s") * 2 + lax.axis_index("c")

    pltpu.sync_copy(inds_hbm.at[wid], idx_v)
    cps = [pltpu.async_copy(w_hbm.at[idx_v.at[j]], rows_v.at[j], sem)
           for j in range(NG)]

    def zero_step(z, _):
        cnt_v[pl.ds(z * 16, 16)] = jnp.zeros((16,), jnp.float32)
        return 0

    lax.fori_loop(0, K // 16, zero_step, 0, unroll=8)
    ones = jnp.ones((16,), jnp.float32)
    for j in range(NG):
        for c in range(GSUB // 16):
            iv = idx_v[j, pl.ds(c * 16, 16)]
            plsc.addupdate_scatter(cnt_v, [iv], ones)
    pltpu.sync_copy(cnt_v, cnt_hbm.at[wid])

    for cp in cps:
        cp.wait()
    pltpu.sync_copy(rows_v, ql_hbm.at[wid])


@functools.cache
def _sc_call():
    return pl.kernel(
        _sc_body,
        out_type=[
            jax.ShapeDtypeStruct((NW, NG, GSUB, D), jnp.float32),
            jax.ShapeDtypeStruct((NW, K), jnp.float32),
        ],
        mesh=plsc.VectorSubcoreMesh(core_axis_name="c",
                                    subcore_axis_name="s"),
        scratch_types=[
            pltpu.VMEM((NG, GSUB), jnp.int32),
            pltpu.VMEM((NG, GSUB, D), jnp.float32),
            pltpu.VMEM((K,), jnp.float32),
            pltpu.SemaphoreType.DMA,
        ],
        compiler_params=pltpu.CompilerParams(needs_layout_passes=False,
                                             use_tc_tiling_on_sc=False),
    )


def _final_body(cnt_ref, md_ref, vq_ref, ent_ref, cm_ref):
    mdsum = jnp.sum(md_ref[...])
    m = mdsum * (1.0 / (B * D))
    vq_ref[0, 0] = m * BETA + m
    cm_ref[0, 0] = mdsum * (1.0 / B)

    def ent_step(j, acc):
        tot = cnt_ref[pl.ds(j * 512, 512)]
        for w in range(1, NW):
            tot = tot + cnt_ref[pl.ds(w * K + j * 512, 512)]
        p = tot * (1.0 / B)
        return acc + jnp.sum(p * jnp.log(p + 1e-10))

    ent = lax.fori_loop(0, K // 512, ent_step, jnp.float32(0.0))
    ent_ref[0, 0] = -ent


_final_call = pl.pallas_call(
    _final_body,
    in_specs=[
        pl.BlockSpec(memory_space=pltpu.VMEM),
        pl.BlockSpec(memory_space=pltpu.VMEM),
    ],
    out_specs=[
        pl.BlockSpec(memory_space=pltpu.SMEM),
        pl.BlockSpec(memory_space=pltpu.SMEM),
        pl.BlockSpec(memory_space=pltpu.SMEM),
    ],
    out_shape=[jax.ShapeDtypeStruct((1, 1), jnp.float32)] * 3,
)


def kernel(latents, W):
    lt = latents.T
    idx3, md3 = _argmin_call(lt, W)
    inds = idx3.reshape(NW, NG, GSUB)
    q4, counts = _sc_call()(inds, W)
    quantized = q4.reshape(B, D)
    vq, ent, cm = _final_call(counts.reshape(NW * K), md3)
    encoding_inds = idx3.reshape(B, 1)
    return (quantized, vq[0, 0], ent[0, 0], encoding_inds, cm[0, 0])


# speedup vs baseline: 1.2147x; 1.1072x over previous
import functools

import jax
import jax.numpy as jnp
from jax import lax
from jax.experimental import pallas as pl
from jax.experimental.pallas import tpu as pltpu
from jax.experimental.pallas import tpu_sc as plsc

B = 16384
K = 8192
D = 32
BETA = 0.25

BN = 512
BKC = 128
GRID = B // BN

NW = 32
CHUNK = B // NW
GSUB = 128
NG = CHUNK // GSUB


def _tree_min(parts):
    while len(parts) > 1:
        parts = [jnp.minimum(parts[i], parts[i + 1])
                 for i in range(0, len(parts), 2)]
    return parts[0]


def _argmin_body(lt_ref, w_ref, idx_ref, md_ref, w2_ref):
    @pl.when(pl.program_id(0) == 0)
    def _():
        wf = w_ref[...]
        w2_ref[...] = jnp.sum(wf * wf, axis=1, keepdims=True)
    lt = lt_ref[...]
    l2 = jnp.sum(lt * lt, axis=0, keepdims=True)
    lt2 = lt + lt
    s_iota = lax.broadcasted_iota(jnp.int32, (8, BN), 0).astype(jnp.float32)
    NGRP = BKC // 8

    def step(c, carry):
        bestv8, besti8 = carry
        wc = w_ref[pl.ds(c * BKC, BKC), :]
        w2 = w2_ref[pl.ds(c * BKC, BKC), :]
        mm2 = lax.dot_general(wc, lt2, (((1,), (0,)), ((), ())),
                              preferred_element_type=jnp.float32)
        dist = (l2 + w2) - mm2
        parts = [lax.slice_in_dim(dist, g * 8, (g + 1) * 8, axis=0)
                 for g in range(NGRP)]
        vals = parts
        idxs = [None] * NGRP
        first = True
        while len(vals) > 1:
            nv, ni = [], []
            for i in range(0, len(vals), 2):
                a, b = vals[i], vals[i + 1]
                le = a <= b
                nv.append(jnp.minimum(a, b))
                if first:
                    ni.append(jnp.where(le, float(i), float(i + 1)))
                else:
                    ni.append(jnp.where(le, idxs[i], idxs[i + 1]))
            vals, idxs, first = nv, ni, False
        r8, gm = vals[0], idxs[0]
        k8 = gm * 8.0 + (s_iota + c * float(BKC))
        upd = r8 < bestv8
        return (jnp.where(upd, r8, bestv8), jnp.where(upd, k8, besti8))

    carry = (jnp.full((8, BN), jnp.inf, jnp.float32),
             jnp.zeros((8, BN), jnp.float32))
    for c in range(K // BKC):
        carry = step(c, carry)
    bestv8, besti8 = carry
    bv = jnp.min(bestv8, axis=0, keepdims=True)
    cand = jnp.where(bestv8 == bv, besti8, float(2 * K))
    bi = jnp.min(cand, axis=0, keepdims=True)
    idx_ref[...] = bi.astype(jnp.int32).reshape(1, 1, BN)
    md_ref[...] = bv.reshape(1, 1, BN)


_argmin_call = pl.pallas_call(
    _argmin_body,
    grid=(GRID,),
    in_specs=[
        pl.BlockSpec((D, BN), lambda i: (0, i)),
        pl.BlockSpec((K, D), lambda i: (0, 0)),
    ],
    out_specs=[
        pl.BlockSpec((1, 1, BN), lambda i: (i, 0, 0)),
        pl.BlockSpec((1, 1, BN), lambda i: (i, 0, 0)),
    ],
    out_shape=[
        jax.ShapeDtypeStruct((GRID, 1, BN), jnp.int32),
        jax.ShapeDtypeStruct((GRID, 1, BN), jnp.float32),
    ],
    scratch_shapes=[pltpu.VMEM((K, 1), jnp.float32)],
    compiler_params=pltpu.CompilerParams(
        dimension_semantics=("arbitrary",)),
)


def _sc_body(inds_hbm, w_hbm, ql_hbm, cnt_hbm,
             idx_v, rows_v, cnt_v, sem):
    wid = lax.axis_index("s") * 2 + lax.axis_index("c")

    pltpu.sync_copy(inds_hbm.at[wid], idx_v)
    cps = [pltpu.async_copy(w_hbm.at[idx_v.at[j]], rows_v.at[j], sem)
           for j in range(NG)]

    def zero_step(z, _):
        cnt_v[pl.ds(z * 16, 16)] = jnp.zeros((16,), jnp.float32)
        return 0

    lax.fori_loop(0, K // 16, zero_step, 0, unroll=8)
    ones = jnp.ones((16,), jnp.float32)
    for j in range(NG):
        for c in range(GSUB // 16):
            iv = idx_v[j, pl.ds(c * 16, 16)]
            plsc.addupdate_scatter(cnt_v, [iv], ones)
    pltpu.sync_copy(cnt_v, cnt_hbm.at[wid])

    for cp in cps:
        cp.wait()
    pltpu.sync_copy(rows_v, ql_hbm.at[wid])


@functools.cache
def _sc_call():
    return pl.kernel(
        _sc_body,
        out_type=[
            jax.ShapeDtypeStruct((NW, NG, GSUB, D), jnp.float32),
            jax.ShapeDtypeStruct((NW, K), jnp.float32),
        ],
        mesh=plsc.VectorSubcoreMesh(core_axis_name="c",
                                    subcore_axis_name="s"),
        scratch_types=[
            pltpu.VMEM((NG, GSUB), jnp.int32),
            pltpu.VMEM((NG, GSUB, D), jnp.float32),
            pltpu.VMEM((K,), jnp.float32),
            pltpu.SemaphoreType.DMA,
        ],
        compiler_params=pltpu.CompilerParams(needs_layout_passes=False,
                                             use_tc_tiling_on_sc=False),
    )


def _final_body(cnt_ref, md_ref, vq_ref, ent_ref, cm_ref):
    mdsum = jnp.sum(md_ref[...])
    m = mdsum * (1.0 / (B * D))
    vq_ref[0, 0] = m * BETA + m
    cm_ref[0, 0] = mdsum * (1.0 / B)

    def ent_step(j, acc):
        tot = cnt_ref[pl.ds(j * 512, 512)]
        for w in range(1, NW):
            tot = tot + cnt_ref[pl.ds(w * K + j * 512, 512)]
        p = tot * (1.0 / B)
        return acc + jnp.sum(p * jnp.log(p + 1e-10))

    ent = lax.fori_loop(0, K // 512, ent_step, jnp.float32(0.0))
    ent_ref[0, 0] = -ent


_final_call = pl.pallas_call(
    _final_body,
    in_specs=[
        pl.BlockSpec(memory_space=pltpu.VMEM),
        pl.BlockSpec(memory_space=pltpu.VMEM),
    ],
    out_specs=[
        pl.BlockSpec(memory_space=pltpu.SMEM),
        pl.BlockSpec(memory_space=pltpu.SMEM),
        pl.BlockSpec(memory_space=pltpu.SMEM),
    ],
    out_shape=[jax.ShapeDtypeStruct((1, 1), jnp.float32)] * 3,
)


def kernel(latents, W):
    lt = latents.T
    idx3, md3 = _argmin_call(lt, W)
    inds = idx3.reshape(NW, NG, GSUB)
    q4, counts = _sc_call()(inds, W)
    quantized = q4.reshape(B, D)
    vq, ent, cm = _final_call(counts.reshape(NW * K), md3)
    encoding_inds = idx3.reshape(B, 1)
    return (quantized, vq[0, 0], ent[0, 0], encoding_inds, cm[0, 0])


# speedup vs baseline: 1.2457x; 1.0255x over previous
import functools

import jax
import jax.numpy as jnp
from jax import lax
from jax.experimental import pallas as pl
from jax.experimental.pallas import tpu as pltpu
from jax.experimental.pallas import tpu_sc as plsc

B = 16384
K = 8192
D = 32
BETA = 0.25

BN = 1024
BKC = 128
GRID = B // BN

NW = 32
CHUNK = B // NW
GSUB = 128
NG = CHUNK // GSUB


def _tree_min(parts):
    while len(parts) > 1:
        parts = [jnp.minimum(parts[i], parts[i + 1])
                 for i in range(0, len(parts), 2)]
    return parts[0]


def _argmin_body(lt_ref, w_ref, idx_ref, md_ref, w2_ref):
    @pl.when(pl.program_id(0) == 0)
    def _():
        wf = w_ref[...]
        w2_ref[...] = jnp.sum(wf * wf, axis=1, keepdims=True)
    lt = lt_ref[...]
    l2 = jnp.sum(lt * lt, axis=0, keepdims=True)
    lt2 = lt + lt
    s_iota = lax.broadcasted_iota(jnp.int32, (8, BN), 0).astype(jnp.float32)
    NGRP = BKC // 8

    def step(c, carry):
        bestv8, besti8 = carry
        wc = w_ref[pl.ds(c * BKC, BKC), :]
        w2 = w2_ref[pl.ds(c * BKC, BKC), :]
        mm2 = lax.dot_general(wc, lt2, (((1,), (0,)), ((), ())),
                              preferred_element_type=jnp.float32)
        dist = (l2 + w2) - mm2
        parts = [lax.slice_in_dim(dist, g * 8, (g + 1) * 8, axis=0)
                 for g in range(NGRP)]
        vals = parts
        idxs = [None] * NGRP
        first = True
        while len(vals) > 1:
            nv, ni = [], []
            for i in range(0, len(vals), 2):
                a, b = vals[i], vals[i + 1]
                le = a <= b
                nv.append(jnp.minimum(a, b))
                if first:
                    ni.append(jnp.where(le, float(i), float(i + 1)))
                else:
                    ni.append(jnp.where(le, idxs[i], idxs[i + 1]))
            vals, idxs, first = nv, ni, False
        r8, gm = vals[0], idxs[0]
        k8 = gm * 8.0 + (s_iota + c * float(BKC))
        upd = r8 < bestv8
        return (jnp.where(upd, r8, bestv8), jnp.where(upd, k8, besti8))

    carry = (jnp.full((8, BN), jnp.inf, jnp.float32),
             jnp.zeros((8, BN), jnp.float32))
    for c in range(K // BKC):
        carry = step(c, carry)
    bestv8, besti8 = carry
    bv = jnp.min(bestv8, axis=0, keepdims=True)
    cand = jnp.where(bestv8 == bv, besti8, float(2 * K))
    bi = jnp.min(cand, axis=0, keepdims=True)
    idx_ref[...] = bi.astype(jnp.int32).reshape(1, 1, BN)
    md_ref[...] = bv.reshape(1, 1, BN)


_argmin_call = pl.pallas_call(
    _argmin_body,
    grid=(GRID,),
    in_specs=[
        pl.BlockSpec((D, BN), lambda i: (0, i)),
        pl.BlockSpec((K, D), lambda i: (0, 0)),
    ],
    out_specs=[
        pl.BlockSpec((1, 1, BN), lambda i: (i, 0, 0)),
        pl.BlockSpec((1, 1, BN), lambda i: (i, 0, 0)),
    ],
    out_shape=[
        jax.ShapeDtypeStruct((GRID, 1, BN), jnp.int32),
        jax.ShapeDtypeStruct((GRID, 1, BN), jnp.float32),
    ],
    scratch_shapes=[pltpu.VMEM((K, 1), jnp.float32)],
    compiler_params=pltpu.CompilerParams(
        dimension_semantics=("arbitrary",)),
)


def _sc_body(inds_hbm, w_hbm, ql_hbm, cnt_hbm,
             idx_v, rows_v, cnt_v, sem):
    wid = lax.axis_index("s") * 2 + lax.axis_index("c")

    pltpu.sync_copy(inds_hbm.at[wid], idx_v)
    cps = [pltpu.async_copy(w_hbm.at[idx_v.at[j]], rows_v.at[j], sem)
           for j in range(NG)]

    def zero_step(z, _):
        cnt_v[pl.ds(z * 16, 16)] = jnp.zeros((16,), jnp.float32)
        return 0

    lax.fori_loop(0, K // 16, zero_step, 0, unroll=8)
    ones = jnp.ones((16,), jnp.float32)
    for j in range(NG):
        for c in range(GSUB // 16):
            iv = idx_v[j, pl.ds(c * 16, 16)]
            plsc.addupdate_scatter(cnt_v, [iv], ones)
    pltpu.sync_copy(cnt_v, cnt_hbm.at[wid])

    for cp in cps:
        cp.wait()
    pltpu.sync_copy(rows_v, ql_hbm.at[wid])


@functools.cache
def _sc_call():
    return pl.kernel(
        _sc_body,
        out_type=[
            jax.ShapeDtypeStruct((NW, NG, GSUB, D), jnp.float32),
            jax.ShapeDtypeStruct((NW, K), jnp.float32),
        ],
        mesh=plsc.VectorSubcoreMesh(core_axis_name="c",
                                    subcore_axis_name="s"),
        scratch_types=[
            pltpu.VMEM((NG, GSUB), jnp.int32),
            pltpu.VMEM((NG, GSUB, D), jnp.float32),
            pltpu.VMEM((K,), jnp.float32),
            pltpu.SemaphoreType.DMA,
        ],
        compiler_params=pltpu.CompilerParams(needs_layout_passes=False,
                                             use_tc_tiling_on_sc=False),
    )


def _final_body(cnt_ref, md_ref, vq_ref, ent_ref, cm_ref):
    mdsum = jnp.sum(md_ref[...])
    m = mdsum * (1.0 / (B * D))
    vq_ref[0, 0] = m * BETA + m
    cm_ref[0, 0] = mdsum * (1.0 / B)

    def ent_step(j, acc):
        tot = cnt_ref[pl.ds(j * 512, 512)]
        for w in range(1, NW):
            tot = tot + cnt_ref[pl.ds(w * K + j * 512, 512)]
        p = tot * (1.0 / B)
        return acc + jnp.sum(p * jnp.log(p + 1e-10))

    ent = lax.fori_loop(0, K // 512, ent_step, jnp.float32(0.0))
    ent_ref[0, 0] = -ent


_final_call = pl.pallas_call(
    _final_body,
    in_specs=[
        pl.BlockSpec(memory_space=pltpu.VMEM),
        pl.BlockSpec(memory_space=pltpu.VMEM),
    ],
    out_specs=[
        pl.BlockSpec(memory_space=pltpu.SMEM),
        pl.BlockSpec(memory_space=pltpu.SMEM),
        pl.BlockSpec(memory_space=pltpu.SMEM),
    ],
    out_shape=[jax.ShapeDtypeStruct((1, 1), jnp.float32)] * 3,
)


def kernel(latents, W):
    lt = latents.T
    idx3, md3 = _argmin_call(lt, W)
    inds = idx3.reshape(NW, NG, GSUB)
    q4, counts = _sc_call()(inds, W)
    quantized = q4.reshape(B, D)
    vq, ent, cm = _final_call(counts.reshape(NW * K), md3)
    encoding_inds = idx3.reshape(B, 1)
    return (quantized, vq[0, 0], ent[0, 0], encoding_inds, cm[0, 0])


# speedup vs baseline: 1.2520x; 1.0050x over previous
import functools

import jax
import jax.numpy as jnp
from jax import lax
from jax.experimental import pallas as pl
from jax.experimental.pallas import tpu as pltpu
from jax.experimental.pallas import tpu_sc as plsc

B = 16384
K = 8192
D = 32
BETA = 0.25

BN = 1024
BKC = 128
GRID = B // BN

NW = 32
CHUNK = B // NW
GSUB = 128
NG = CHUNK // GSUB


def _tree_min(parts):
    while len(parts) > 1:
        parts = [jnp.minimum(parts[i], parts[i + 1])
                 for i in range(0, len(parts), 2)]
    return parts[0]


def _argmin_body(lt_ref, w_ref, idx_ref, md_ref, w2_ref):
    @pl.when(pl.program_id(0) == 0)
    def _():
        wf = w_ref[...]
        w2_ref[...] = jnp.sum(wf * wf, axis=1, keepdims=True)
    lt = lt_ref[...]
    l2 = jnp.sum(lt * lt, axis=0, keepdims=True)
    lt2 = lt + lt
    s_iota = lax.broadcasted_iota(jnp.int32, (8, BN), 0).astype(jnp.float32)
    NGRP = BKC // 8

    def step(c, carry):
        bestv8, besti8 = carry
        wc = w_ref[pl.ds(c * BKC, BKC), :]
        w2 = w2_ref[pl.ds(c * BKC, BKC), :]
        mm2 = lax.dot_general(wc, lt2, (((1,), (0,)), ((), ())),
                              preferred_element_type=jnp.float32)
        dist = (l2 + w2) - mm2
        parts = [lax.slice_in_dim(dist, g * 8, (g + 1) * 8, axis=0)
                 for g in range(NGRP)]
        vals = parts
        idxs = [None] * NGRP
        first = True
        while len(vals) > 1:
            nv, ni = [], []
            for i in range(0, len(vals), 2):
                a, b = vals[i], vals[i + 1]
                le = a <= b
                nv.append(jnp.minimum(a, b))
                if first:
                    ni.append(jnp.where(le, float(i), float(i + 1)))
                else:
                    ni.append(jnp.where(le, idxs[i], idxs[i + 1]))
            vals, idxs, first = nv, ni, False
        r8, gm = vals[0], idxs[0]
        k8 = gm * 8.0 + (s_iota + c * float(BKC))
        upd = r8 < bestv8
        return (jnp.where(upd, r8, bestv8), jnp.where(upd, k8, besti8))

    carry = (jnp.full((8, BN), jnp.inf, jnp.float32),
             jnp.zeros((8, BN), jnp.float32))
    for c in range(K // BKC):
        carry = step(c, carry)
    bestv8, besti8 = carry
    bv = jnp.min(bestv8, axis=0, keepdims=True)
    cand = jnp.where(bestv8 == bv, besti8, float(2 * K))
    bi = jnp.min(cand, axis=0, keepdims=True)
    idx_ref[...] = bi.astype(jnp.int32).reshape(1, 1, BN)
    md_ref[...] = bv.reshape(1, 1, BN)


_argmin_call = pl.pallas_call(
    _argmin_body,
    grid=(GRID,),
    in_specs=[
        pl.BlockSpec((D, BN), lambda i: (0, i)),
        pl.BlockSpec((K, D), lambda i: (0, 0)),
    ],
    out_specs=[
        pl.BlockSpec((1, 1, BN), lambda i: (i, 0, 0)),
        pl.BlockSpec((1, 1, BN), lambda i: (i, 0, 0)),
    ],
    out_shape=[
        jax.ShapeDtypeStruct((GRID, 1, BN), jnp.int32),
        jax.ShapeDtypeStruct((GRID, 1, BN), jnp.float32),
    ],
    scratch_shapes=[pltpu.VMEM((K, 1), jnp.float32)],
    compiler_params=pltpu.CompilerParams(
        dimension_semantics=("parallel",)),
)


def _sc_body(inds_hbm, w_hbm, ql_hbm, cnt_hbm,
             idx_v, rows_v, cnt_v, sem):
    wid = lax.axis_index("s") * 2 + lax.axis_index("c")

    pltpu.sync_copy(inds_hbm.at[wid], idx_v)
    cps = [pltpu.async_copy(w_hbm.at[idx_v.at[j]], rows_v.at[j], sem)
           for j in range(NG)]

    def zero_step(z, _):
        cnt_v[pl.ds(z * 16, 16)] = jnp.zeros((16,), jnp.float32)
        return 0

    lax.fori_loop(0, K // 16, zero_step, 0, unroll=8)
    ones = jnp.ones((16,), jnp.float32)
    for j in range(NG):
        for c in range(GSUB // 16):
            iv = idx_v[j, pl.ds(c * 16, 16)]
            plsc.addupdate_scatter(cnt_v, [iv], ones)
    pltpu.sync_copy(cnt_v, cnt_hbm.at[wid])

    for cp in cps:
        cp.wait()
    pltpu.sync_copy(rows_v, ql_hbm.at[wid])


@functools.cache
def _sc_call():
    return pl.kernel(
        _sc_body,
        out_type=[
            jax.ShapeDtypeStruct((NW, NG, GSUB, D), jnp.float32),
            jax.ShapeDtypeStruct((NW, K), jnp.float32),
        ],
        mesh=plsc.VectorSubcoreMesh(core_axis_name="c",
                                    subcore_axis_name="s"),
        scratch_types=[
            pltpu.VMEM((NG, GSUB), jnp.int32),
            pltpu.VMEM((NG, GSUB, D), jnp.float32),
            pltpu.VMEM((K,), jnp.float32),
            pltpu.SemaphoreType.DMA,
        ],
        compiler_params=pltpu.CompilerParams(needs_layout_passes=False,
                                             use_tc_tiling_on_sc=False),
    )


def _final_body(cnt_ref, md_ref, vq_ref, ent_ref, cm_ref):
    mdsum = jnp.sum(md_ref[...])
    m = mdsum * (1.0 / (B * D))
    vq_ref[0, 0] = m * BETA + m
    cm_ref[0, 0] = mdsum * (1.0 / B)

    def ent_step(j, acc):
        tot = cnt_ref[pl.ds(j * 512, 512)]
        for w in range(1, NW):
            tot = tot + cnt_ref[pl.ds(w * K + j * 512, 512)]
        p = tot * (1.0 / B)
        return acc + jnp.sum(p * jnp.log(p + 1e-10))

    ent = lax.fori_loop(0, K // 512, ent_step, jnp.float32(0.0))
    ent_ref[0, 0] = -ent


_final_call = pl.pallas_call(
    _final_body,
    in_specs=[
        pl.BlockSpec(memory_space=pltpu.VMEM),
        pl.BlockSpec(memory_space=pltpu.VMEM),
    ],
    out_specs=[
        pl.BlockSpec(memory_space=pltpu.SMEM),
        pl.BlockSpec(memory_space=pltpu.SMEM),
        pl.BlockSpec(memory_space=pltpu.SMEM),
    ],
    out_shape=[jax.ShapeDtypeStruct((1, 1), jnp.float32)] * 3,
)


def kernel(latents, W):
    lt = latents.T
    idx3, md3 = _argmin_call(lt, W)
    inds = idx3.reshape(NW, NG, GSUB)
    q4, counts = _sc_call()(inds, W)
    quantized = q4.reshape(B, D)
    vq, ent, cm = _final_call(counts.reshape(NW * K), md3)
    encoding_inds = idx3.reshape(B, 1)
    return (quantized, vq[0, 0], ent[0, 0], encoding_inds, cm[0, 0])
